# baseline (speedup 1.0000x reference)
import jax
import jax.numpy as jnp
from jax.experimental import pallas as pl
from jax.experimental.pallas import tpu as pltpu

_GRID = (16, 16)
_G = _GRID[0] * _GRID[1]


def _build_mt(pos, wt_ref):
    p0 = pos[:, 0:1] * (_GRID[0] / 2) + (_GRID[0] / 2)
    p1 = pos[:, 1:2] * (_GRID[1] / 2) + (_GRID[1] / 2)
    gi = jax.lax.broadcasted_iota(jnp.int32, (1, _G), 1)
    row = (gi // _GRID[1]).astype(jnp.float32)
    col = (gi % _GRID[1]).astype(jnp.float32)
    a = jnp.maximum(1.0 - jnp.abs(p0 - row), 0.0)
    a *= jnp.maximum(1.0 - jnp.abs(p1 - col), 0.0)
    return jnp.dot(
        a, wt_ref[:], preferred_element_type=jnp.float32
    ).astype(jnp.bfloat16)


def _merger_kernel(pos_cur_ref, pos_nxt_ref, x_ref, wt_ref, out_ref, mt_ref):
    b = pl.program_id(0)
    nb = pl.num_programs(0)

    @pl.when(b == 0)
    def _peel():
        mt_ref[0] = _build_mt(pos_cur_ref[0], wt_ref)

    cur = jax.lax.rem(b, 2)
    out_ref[0] = jax.lax.dot_general(
        mt_ref[cur],
        x_ref[0],
        (((0,), (0,)), ((), ())),
        preferred_element_type=jnp.float32,
    )

    @pl.when(b + 1 < nb)
    def _prefetch():
        mt_ref[1 - cur] = _build_mt(pos_nxt_ref[0], wt_ref)


@jax.jit
def kernel(x, positions, grid_weights):
    B, C, T = x.shape
    M = grid_weights.shape[0]
    grid = (B,)
    out = pl.pallas_call(
        _merger_kernel,
        grid=grid,
        in_specs=[
            pl.BlockSpec((1, C, 2), lambda b: (b, 0, 0)),
            pl.BlockSpec((1, C, 2), lambda b: (jnp.minimum(b + 1, B - 1), 0, 0)),
            pl.BlockSpec((1, C, T), lambda b: (b, 0, 0)),
            pl.BlockSpec((_G, M), lambda b: (0, 0)),
        ],
        out_specs=pl.BlockSpec((1, M, T), lambda b: (b, 0, 0)),
        out_shape=jax.ShapeDtypeStruct((B, M, T), jnp.float32),
        scratch_shapes=[
            pltpu.VMEM((2, C, M), jnp.bfloat16),
        ],
        compiler_params=pltpu.CompilerParams(
            dimension_semantics=("arbitrary",),
        ),
    )(positions, positions, x, grid_weights.T)
    return out


# speedup vs baseline: 1.0281x; 1.0281x over previous
import jax
import jax.numpy as jnp
from jax.experimental import pallas as pl
from jax.experimental.pallas import tpu as pltpu

_GRID = (16, 16)
_G = _GRID[0] * _GRID[1]


def _merger_kernel(pos_ref, x_ref, wt_ref, out_ref):
    pos = pos_ref[0]
    p0 = pos[:, 0:1] * (_GRID[0] / 2) + (_GRID[0] / 2)
    p1 = pos[:, 1:2] * (_GRID[1] / 2) + (_GRID[1] / 2)
    gi = jax.lax.broadcasted_iota(jnp.int32, (1, _G), 1)
    row = (gi // _GRID[1]).astype(jnp.float32)
    col = (gi % _GRID[1]).astype(jnp.float32)
    a = jnp.maximum(1.0 - jnp.abs(p0 - row), 0.0)
    a *= jnp.maximum(1.0 - jnp.abs(p1 - col), 0.0)
    mt = jnp.dot(a, wt_ref[:], preferred_element_type=jnp.float32)
    out_ref[0] = jax.lax.dot_general(
        mt.astype(jnp.bfloat16),
        x_ref[0].astype(jnp.bfloat16),
        (((0,), (0,)), ((), ())),
        preferred_element_type=jnp.float32,
    )


@jax.jit
def kernel(x, positions, grid_weights):
    B, C, T = x.shape
    M = grid_weights.shape[0]
    out = pl.pallas_call(
        _merger_kernel,
        grid=(B,),
        in_specs=[
            pl.BlockSpec((1, C, 2), lambda b: (b, 0, 0)),
            pl.BlockSpec((1, C, T), lambda b: (b, 0, 0)),
            pl.BlockSpec((_G, M), lambda b: (0, 0)),
        ],
        out_specs=pl.BlockSpec((1, M, T), lambda b: (b, 0, 0)),
        out_shape=jax.ShapeDtypeStruct((B, M, T), jnp.float32),
        compiler_params=pltpu.CompilerParams(
            dimension_semantics=("arbitrary",),
        ),
    )(positions, x, grid_weights.T)
    return out
